# baseline (speedup 1.0000x reference)
import functools

import jax
import jax.numpy as jnp
from jax.experimental import pallas as pl
from jax.experimental.pallas import tpu as pltpu

NUM_EXPERTS = 8
TOP_K = 2
HIDDEN = 1024
INTER = 2048
TOKENS = 2048

T_TILE = 256
N_TTILES = TOKENS // T_TILE


def _moe_body(x_ref, rl_ref, w13_ref, w2_ref, out_ref):
    e = pl.program_id(0)
    t = pl.program_id(1)

    logits = rl_ref[...].astype(jnp.float32)
    m = jnp.max(logits, axis=-1, keepdims=True)
    p = jnp.exp(logits - m)
    probs = p / jnp.sum(p, axis=-1, keepdims=True)

    idx = jax.lax.broadcasted_iota(jnp.int32, probs.shape, 1)
    p1 = jnp.max(probs, axis=-1, keepdims=True)
    is1 = probs == p1
    id1 = jnp.min(jnp.where(is1, idx, NUM_EXPERTS), axis=-1, keepdims=True)
    probs2 = jnp.where(idx == id1, -jnp.inf, probs)
    p2 = jnp.max(probs2, axis=-1, keepdims=True)
    is2 = probs2 == p2
    id2 = jnp.min(jnp.where(is2, idx, NUM_EXPERTS), axis=-1, keepdims=True)

    denom = p1 + p2
    w1 = p1 / denom
    w2 = p2 / denom
    combine = jnp.where(id1 == e, w1, 0.0) + jnp.where(id2 == e, w2, 0.0)

    xb = x_ref[...].astype(jnp.bfloat16)
    w13 = w13_ref[0]
    gu = jnp.dot(xb, w13, preferred_element_type=jnp.float32)
    gate = gu[:, :INTER]
    up = gu[:, INTER:]
    h = (gate * jax.nn.sigmoid(gate)) * up
    y = jnp.dot(h.astype(jnp.bfloat16), w2_ref[0],
                preferred_element_type=jnp.float32)
    part = combine * y

    sl = pl.ds(t * T_TILE, T_TILE)

    @pl.when(e == 0)
    def _():
        out_ref[sl, :] = part

    @pl.when(e > 0)
    def _():
        out_ref[sl, :] += part


@jax.jit
def kernel(x, router_logits, w13_weight, w2_weight):
    w13_t = jnp.transpose(w13_weight, (0, 2, 1)).astype(jnp.bfloat16)
    w2_t = jnp.transpose(w2_weight, (0, 2, 1)).astype(jnp.bfloat16)

    grid = (NUM_EXPERTS, N_TTILES)
    out = pl.pallas_call(
        _moe_body,
        grid=grid,
        in_specs=[
            pl.BlockSpec((T_TILE, HIDDEN), lambda e, t: (t, 0)),
            pl.BlockSpec((T_TILE, NUM_EXPERTS), lambda e, t: (t, 0)),
            pl.BlockSpec((1, HIDDEN, 2 * INTER), lambda e, t: (e, 0, 0)),
            pl.BlockSpec((1, INTER, HIDDEN), lambda e, t: (e, 0, 0)),
        ],
        out_specs=pl.BlockSpec((TOKENS, HIDDEN), lambda e, t: (0, 0)),
        out_shape=jax.ShapeDtypeStruct((TOKENS, HIDDEN), jnp.float32),
        compiler_params=pltpu.CompilerParams(
            dimension_semantics=("arbitrary", "arbitrary"),
        ),
    )(x, router_logits, w13_t, w2_t)
    return out
